# baseline (speedup 1.0000x reference)
import functools
import math

import jax
import jax.numpy as jnp
from jax import lax
from jax.experimental import pallas as pl
from jax.experimental.pallas import tpu as pltpu
from jax.experimental.pallas import tpu_sc as plsc

D_MODEL = 64
SCALE = math.sqrt(D_MODEL)

NC = 2
NS = 16
NW = NC * NS
LANES = 16

CHUNK = 128


@functools.lru_cache(maxsize=None)
def _build(n_idx: int, vocab: int, d: int):
    assert n_idx % (NW * CHUNK) == 0
    b_per_w = n_idx // NW
    n_chunks = b_per_w // CHUNK
    mesh = plsc.VectorSubcoreMesh(core_axis_name="c", subcore_axis_name="s")

    @functools.partial(
        pl.kernel,
        out_type=jax.ShapeDtypeStruct((n_idx, d), jnp.float32),
        mesh=mesh,
        scratch_types=[
            pltpu.VMEM((CHUNK,), jnp.int32),
            pltpu.VMEM((CHUNK, d), jnp.float32),
            pltpu.SemaphoreType.DMA,
        ],
        compiler_params=pltpu.CompilerParams(use_tc_tiling_on_sc=False),
    )
    def emb_kernel(x_hbm, table_hbm, out_hbm, idx_v, rows_v, sem):
        wid = lax.axis_index("s") * NC + lax.axis_index("c")
        base = wid * b_per_w

        def chunk_body(c, carry):
            off = base + c * CHUNK
            pltpu.sync_copy(x_hbm.at[pl.ds(off, CHUNK)], idx_v)
            pltpu.async_copy(table_hbm.at[idx_v], rows_v, sem).wait()

            def scale_row(r, carry2):
                for j in range(d // LANES):
                    sl = pl.ds(j * LANES, LANES)
                    rows_v[r, sl] = rows_v[r, sl] * SCALE
                return carry2

            lax.fori_loop(0, CHUNK, scale_row, 0, unroll=2)
            pltpu.sync_copy(rows_v, out_hbm.at[pl.ds(off, CHUNK)])
            return carry

        lax.fori_loop(0, n_chunks, chunk_body, 0)

    return emb_kernel


def kernel(x, table):
    b, s = x.shape
    vocab, d = table.shape
    flat_idx = x.reshape(b * s).astype(jnp.int32)
    out = _build(b * s, vocab, d)(flat_idx, table)
    return out.reshape(b, s, d)


# speedup vs baseline: 1.2253x; 1.2253x over previous
import functools
import math

import jax
import jax.numpy as jnp
from jax import lax
from jax.experimental import pallas as pl
from jax.experimental.pallas import tpu as pltpu
from jax.experimental.pallas import tpu_sc as plsc

D_MODEL = 64
SCALE = math.sqrt(D_MODEL)

NC = 2
NS = 16
NW = NC * NS
LANES = 16

CHUNK = 128
NBUF = 8
LOOKAHEAD = 4


@functools.lru_cache(maxsize=None)
def _build(n_idx: int, vocab: int, d: int):
    assert n_idx % (NW * CHUNK) == 0
    b_per_w = n_idx // NW
    n_chunks = b_per_w // CHUNK
    assert n_chunks % NBUF == 0 and n_chunks > NBUF
    mesh = plsc.VectorSubcoreMesh(core_axis_name="c", subcore_axis_name="s")

    @functools.partial(
        pl.kernel,
        out_type=jax.ShapeDtypeStruct((n_idx, d), jnp.float32),
        mesh=mesh,
        scratch_types=[
            pltpu.VMEM((n_chunks, CHUNK), jnp.int32),
            pltpu.VMEM((NBUF, CHUNK, d), jnp.float32),
        ]
        + [pltpu.SemaphoreType.DMA] * (2 * NBUF),
        compiler_params=pltpu.CompilerParams(use_tc_tiling_on_sc=False),
    )
    def emb_kernel(x_hbm, table_hbm, out_hbm, idx_all, rows_v, *sems):
        gsem = sems[:NBUF]
        wsem = sems[NBUF:]
        wid = lax.axis_index("s") * NC + lax.axis_index("c")
        base = wid * b_per_w

        pltpu.sync_copy(x_hbm.at[wid], idx_all)

        def issue_gather(n, s):
            pltpu.async_copy(table_hbm.at[idx_all.at[n]], rows_v.at[s], gsem[s])

        def wait_gather(n, s):
            pltpu.make_async_copy(
                table_hbm.at[idx_all.at[n]], rows_v.at[s], gsem[s]
            ).wait()

        def issue_write(c, s):
            pltpu.async_copy(
                rows_v.at[s], out_hbm.at[pl.ds(base + c * CHUNK, CHUNK)], wsem[s]
            )

        def wait_write(c, s):
            pltpu.make_async_copy(
                rows_v.at[s], out_hbm.at[pl.ds(base + c * CHUNK, CHUNK)], wsem[s]
            ).wait()

        for b in range(LOOKAHEAD):
            issue_gather(b, b)

        def outer(o, carry):
            for b in range(NBUF):
                c = o * NBUF + b
                s = b
                wait_gather(c, s)

                def scale_row(r, carry2):
                    for j in range(d // LANES):
                        sl = pl.ds(j * LANES, LANES)
                        rows_v[s, r, sl] = rows_v[s, r, sl] * SCALE
                    return carry2

                lax.fori_loop(0, CHUNK, scale_row, 0, unroll=4)
                issue_write(c, s)

                nxt = c + LOOKAHEAD
                s_n = (b + LOOKAHEAD) % NBUF
                fresh = c < NBUF - LOOKAHEAD

                @pl.when(jnp.logical_and(nxt < n_chunks, jnp.logical_not(fresh)))
                def _():
                    wait_write(nxt - NBUF, s_n)
                    issue_gather(nxt, s_n)

                @pl.when(jnp.logical_and(nxt < n_chunks, fresh))
                def _():
                    issue_gather(nxt, s_n)

            return carry

        lax.fori_loop(0, n_chunks // NBUF, outer, 0)

        for b in range(NBUF):
            wait_write(n_chunks - NBUF + b, b)

    return emb_kernel


def kernel(x, table):
    b, s = x.shape
    vocab, d = table.shape
    n_idx = b * s
    b_per_w = n_idx // NW
    flat_idx = x.reshape(NW, b_per_w // CHUNK, CHUNK).astype(jnp.int32)
    out = _build(n_idx, vocab, d)(flat_idx, table)
    return out.reshape(b, s, d)


# speedup vs baseline: 1.2265x; 1.0010x over previous
import functools
import math

import jax
import jax.numpy as jnp
from jax import lax
from jax.experimental import pallas as pl
from jax.experimental.pallas import tpu as pltpu
from jax.experimental.pallas import tpu_sc as plsc

D_MODEL = 64
SCALE = math.sqrt(D_MODEL)

NC = 2
NS = 16
NW = NC * NS
LANES = 16

TOK_PER_CHUNK = 2
NBUF = 8
LOOKAHEAD = 4


@functools.lru_cache(maxsize=None)
def _build(n_tok: int, seq: int, vocab: int, d: int):
    assert n_tok % (NW * TOK_PER_CHUNK) == 0
    t_per_w = n_tok // NW
    n_chunks = t_per_w // TOK_PER_CHUNK
    rows = TOK_PER_CHUNK * seq
    assert rows <= 128 and n_chunks % NBUF == 0 and n_chunks > NBUF
    mesh = plsc.VectorSubcoreMesh(core_axis_name="c", subcore_axis_name="s")

    @functools.partial(
        pl.kernel,
        out_type=jax.ShapeDtypeStruct((n_tok, seq, d), jnp.float32),
        mesh=mesh,
        scratch_types=[
            pltpu.VMEM((n_chunks, rows), jnp.int32),
            pltpu.VMEM((NBUF, rows, d), jnp.float32),
        ]
        + [pltpu.SemaphoreType.DMA] * (2 * NBUF),
        compiler_params=pltpu.CompilerParams(use_tc_tiling_on_sc=False),
    )
    def emb_kernel(x_hbm, table_hbm, out_hbm, idx_all, rows_v, *sems):
        gsem = sems[:NBUF]
        wsem = sems[NBUF:]
        wid = lax.axis_index("s") * NC + lax.axis_index("c")
        tok0 = wid * t_per_w

        pltpu.sync_copy(x_hbm.at[wid], idx_all)

        def issue_gather(n, s):
            pltpu.async_copy(table_hbm.at[idx_all.at[n]], rows_v.at[s], gsem[s])

        def wait_gather(n, s):
            pltpu.make_async_copy(
                table_hbm.at[idx_all.at[n]], rows_v.at[s], gsem[s]
            ).wait()

        def issue_write(c, s):
            for t in range(TOK_PER_CHUNK):
                pltpu.async_copy(
                    rows_v.at[s].at[pl.ds(t * seq, seq)],
                    out_hbm.at[tok0 + c * TOK_PER_CHUNK + t],
                    wsem[s],
                )

        def wait_write(c, s):
            for t in range(TOK_PER_CHUNK):
                pltpu.make_async_copy(
                    rows_v.at[s].at[pl.ds(t * seq, seq)],
                    out_hbm.at[tok0 + c * TOK_PER_CHUNK + t],
                    wsem[s],
                ).wait()

        for b in range(LOOKAHEAD):
            issue_gather(b, b)

        def outer(o, carry):
            for b in range(NBUF):
                c = o * NBUF + b
                s = b
                wait_gather(c, s)

                def scale_row(r, carry2):
                    for j in range(d // LANES):
                        sl = pl.ds(j * LANES, LANES)
                        rows_v[s, r, sl] = rows_v[s, r, sl] * SCALE
                    return carry2

                lax.fori_loop(0, rows, scale_row, 0, unroll=4)
                issue_write(c, s)

                nxt = c + LOOKAHEAD
                s_n = (b + LOOKAHEAD) % NBUF
                fresh = c < NBUF - LOOKAHEAD

                @pl.when(jnp.logical_and(nxt < n_chunks, jnp.logical_not(fresh)))
                def _():
                    wait_write(nxt - NBUF, s_n)
                    issue_gather(nxt, s_n)

                @pl.when(jnp.logical_and(nxt < n_chunks, fresh))
                def _():
                    issue_gather(nxt, s_n)

            return carry

        lax.fori_loop(0, n_chunks // NBUF, outer, 0)

        for b in range(NBUF):
            wait_write(n_chunks - NBUF + b, b)

    return emb_kernel


def kernel(x, table):
    n_tok, seq = x.shape
    vocab, d = table.shape
    t_per_w = n_tok // NW
    idx = x.reshape(NW, t_per_w // TOK_PER_CHUNK, TOK_PER_CHUNK * seq).astype(
        jnp.int32
    )
    return _build(n_tok, seq, vocab, d)(idx, table)
